# baseline (speedup 1.0000x reference)
import jax
import jax.numpy as jnp
from jax import lax
from jax.experimental import pallas as pl
from jax.experimental.pallas import tpu as pltpu

_B = 64
_N = 325
_IN = 2
_OUT = 1
_H = 64
_ORDER = 2
_HORIZON = 12
_SEQ = 12
_K = 5


def _dcrnn_body(xin_ref, s0_ref, s1_ref,
                e0Wg, e0bg, e0Wc, e0bc,
                e1Wg, e1bg, e1Wc, e1bc,
                d0Wg, d0bg, d0Wc, d0bc,
                d1Wg, d1bg, d1Wc, d1bc,
                pWc, pWr, pb, out_ref):
    bf = jnp.bfloat16
    s0 = s0_ref[...].astype(bf)
    s1 = s1_ref[...].astype(bf)

    def gconv(cat, W_ref, b_ref):
        cat = cat.astype(bf)
        acc = jnp.dot(cat, W_ref[0], preferred_element_type=jnp.float32)
        acc = acc + b_ref[...]
        k = 1
        for s in (s0, s1):
            xk = cat
            for _ in range(_ORDER):
                xk = jnp.dot(s, xk,
                             preferred_element_type=jnp.float32).astype(bf)
                acc = acc + jnp.dot(xk, W_ref[k],
                                    preferred_element_type=jnp.float32)
                k += 1
        return acc

    def cell(x, h, Wg, bg, Wc, bc):
        ru = jax.nn.sigmoid(gconv(jnp.concatenate([x, h], axis=1), Wg, bg))
        r = ru[:, :_H]
        u = ru[:, _H:]
        c = jnp.tanh(gconv(jnp.concatenate([x, r * h], axis=1), Wc, bc))
        return u * h + (1.0 - u) * c

    z = jnp.zeros((_N, _H), dtype=jnp.float32)

    def enc_step(t, hs):
        h0, h1 = hs
        x = xin_ref[0, t]
        h0 = cell(x, h0, e0Wg, e0bg, e0Wc, e0bc)
        h1 = cell(h0, h1, e1Wg, e1bg, e1Wc, e1bc)
        return (h0, h1)

    h0, h1 = lax.fori_loop(0, _SEQ, enc_step, (z, z))

    def dec_step(t, carry):
        h0, h1, x = carry
        h0 = cell(x, h0, d0Wg, d0bg, d0Wc, d0bc)
        h1 = cell(h0, h1, d1Wg, d1bg, d1Wc, d1bc)
        h1b = h1.astype(bf)
        y_col = jnp.dot(h1b, pWc[...], preferred_element_type=jnp.float32)
        y_col = y_col + pb[...]
        y_row = lax.dot_general(pWr[...], h1b, (((1,), (1,)), ((), ())),
                                preferred_element_type=jnp.float32)
        out_ref[0, t] = y_row + pb[...].T
        return (h0, h1, y_col)

    x0 = jnp.zeros((_N, _OUT), dtype=jnp.float32)
    lax.fori_loop(0, _HORIZON, dec_step, (h0, h1, x0))


def _perm(W):
    C = W.shape[0] // _K
    return W.reshape(C, _K, W.shape[1]).transpose(1, 0, 2)


def kernel(inputs, s0, s1,
           enc0_Wg, enc0_bg, enc0_Wc, enc0_bc,
           enc1_Wg, enc1_bg, enc1_Wc, enc1_bc,
           dec0_Wg, dec0_bg, dec0_Wc, dec0_bc,
           dec1_Wg, dec1_bg, dec1_Wc, dec1_bc,
           proj_W, proj_b):
    xin = inputs.transpose(0, 3, 2, 1)
    Ws = []
    for Wg, bg, Wc, bc in ((enc0_Wg, enc0_bg, enc0_Wc, enc0_bc),
                           (enc1_Wg, enc1_bg, enc1_Wc, enc1_bc),
                           (dec0_Wg, dec0_bg, dec0_Wc, dec0_bc),
                           (dec1_Wg, dec1_bg, dec1_Wc, dec1_bc)):
        Ws += [_perm(Wg).astype(jnp.bfloat16), bg.reshape(1, -1),
               _perm(Wc).astype(jnp.bfloat16), bc.reshape(1, -1)]
    pWc = proj_W.T.astype(jnp.bfloat16)
    pWr = proj_W.astype(jnp.bfloat16)
    pb = proj_b.reshape(1, -1)

    def w_spec(a):
        return pl.BlockSpec(a.shape, lambda i: (0,) * a.ndim)

    operands = [xin, s0, s1] + Ws + [pWc, pWr, pb]
    in_specs = [pl.BlockSpec((1, _SEQ, _N, _IN), lambda i: (i, 0, 0, 0))]
    in_specs += [w_spec(a) for a in operands[1:]]

    out = pl.pallas_call(
        _dcrnn_body,
        grid=(_B,),
        in_specs=in_specs,
        out_specs=pl.BlockSpec((1, _HORIZON, _OUT, _N),
                               lambda i: (i, 0, 0, 0)),
        out_shape=jax.ShapeDtypeStruct((_B, _HORIZON, _OUT, _N), jnp.float32),
        compiler_params=pltpu.CompilerParams(
            dimension_semantics=("parallel",)),
    )(*operands)
    return out.transpose(0, 2, 3, 1)


# speedup vs baseline: 2.1483x; 2.1483x over previous
import jax
import jax.numpy as jnp
from jax import lax
from jax.experimental import pallas as pl
from jax.experimental.pallas import tpu as pltpu

_B = 64
_N = 325
_IN = 2
_OUT = 1
_H = 64
_ORDER = 2
_HORIZON = 12
_SEQ = 12
_K = 5
_P = 4
_G = _B // _P


def _dcrnn_body(xin_ref, s0_ref, s1_ref, *refs):
    L = [refs[9 * i:9 * (i + 1)] for i in range(4)]
    pcol, prow, pb, out_ref = refs[36:]
    bf = jnp.bfloat16
    s0 = s0_ref[...].astype(bf)
    s1 = s1_ref[...].astype(bf)

    def chain(v):
        v0 = v.astype(bf)
        out = [v0]
        for s in (s0, s1):
            xk = v0
            for _ in range(_ORDER):
                xk = jnp.dot(s, xk,
                             preferred_element_type=jnp.float32).astype(bf)
                out.append(xk)
        return out

    def msum(ks, W_ref, acc):
        for k in range(_K):
            acc = acc + jnp.dot(ks[k], W_ref[k],
                                preferred_element_type=jnp.float32)
        return acc

    def cell(x, h, lw):
        Wrx, Wux, Wcx, Wrh, Wuh, Wch, br, bu, bc = lw
        Xk = chain(x)
        Hk = chain(h)
        r = jax.nn.sigmoid(msum(Hk, Wrh, msum(Xk, Wrx, br[...])))
        u = jax.nn.sigmoid(msum(Hk, Wuh, msum(Xk, Wux, bu[...])))
        Rk = chain(r * h)
        c = jnp.tanh(msum(Rk, Wch, msum(Xk, Wcx, bc[...])))
        return u * h + (1.0 - u) * c

    z = jnp.zeros((_N, _P * _H), dtype=jnp.float32)

    def enc_step(t, hs):
        h0, h1 = hs
        x = xin_ref[0, t]
        h0 = cell(x, h0, L[0])
        h1 = cell(h0, h1, L[1])
        return (h0, h1)

    h0, h1 = lax.fori_loop(0, _SEQ, enc_step, (z, z))

    def dec_step(t, carry):
        h0, h1, x = carry
        h0 = cell(x, h0, L[2])
        h1 = cell(h0, h1, L[3])
        h1b = h1.astype(bf)
        y_col = jnp.dot(h1b, pcol[...],
                        preferred_element_type=jnp.float32) + pb[0, 0]
        y_row = lax.dot_general(prow[...], h1b, (((1,), (1,)), ((), ())),
                                preferred_element_type=jnp.float32)
        out_ref[0, t] = y_row + pb[0, 0]
        return (h0, h1, y_col)

    x0 = jnp.zeros((_N, _P * _OUT), dtype=jnp.float32)
    lax.fori_loop(0, _HORIZON, dec_step, (h0, h1, x0))


def _bd(M):
    K, a, b = M.shape
    out = jnp.einsum('ij,kab->kiajb', jnp.eye(_P, dtype=M.dtype), M)
    return out.reshape(K, _P * a, _P * b).astype(jnp.bfloat16)


def _layer_weights(Wg, bg, Wc, bc, cin):
    C = cin + _H
    pg = Wg.reshape(C, _K, 2 * _H).transpose(1, 0, 2)
    pc = Wc.reshape(C, _K, _H).transpose(1, 0, 2)
    Wrx = _bd(pg[:, :cin, :_H])
    Wux = _bd(pg[:, :cin, _H:])
    Wcx = _bd(pc[:, :cin, :])
    Wrh = _bd(pg[:, cin:, :_H])
    Wuh = _bd(pg[:, cin:, _H:])
    Wch = _bd(pc[:, cin:, :])
    br = jnp.tile(bg[:_H], _P).reshape(1, -1)
    bu = jnp.tile(bg[_H:], _P).reshape(1, -1)
    bcc = jnp.tile(bc, _P).reshape(1, -1)
    return [Wrx, Wux, Wcx, Wrh, Wuh, Wch, br, bu, bcc]


def kernel(inputs, s0, s1,
           enc0_Wg, enc0_bg, enc0_Wc, enc0_bc,
           enc1_Wg, enc1_bg, enc1_Wc, enc1_bc,
           dec0_Wg, dec0_bg, dec0_Wc, dec0_bc,
           dec1_Wg, dec1_bg, dec1_Wc, dec1_bc,
           proj_W, proj_b):
    xin = (inputs.reshape(_G, _P, _IN, _N, _SEQ)
           .transpose(0, 4, 3, 1, 2).reshape(_G, _SEQ, _N, _P * _IN))
    Ws = []
    for Wg, bg, Wc, bc, cin in ((enc0_Wg, enc0_bg, enc0_Wc, enc0_bc, _IN),
                                (enc1_Wg, enc1_bg, enc1_Wc, enc1_bc, _H),
                                (dec0_Wg, dec0_bg, dec0_Wc, dec0_bc, _OUT),
                                (dec1_Wg, dec1_bg, dec1_Wc, dec1_bc, _H)):
        Ws += _layer_weights(Wg, bg, Wc, bc, cin)
    eyeP = jnp.eye(_P, dtype=jnp.float32)
    pcol = jnp.einsum('ij,ao->iaj', eyeP,
                      proj_W.T).reshape(_P * _H, _P).astype(jnp.bfloat16)
    prow = jnp.einsum('ij,oa->ija', eyeP,
                      proj_W).reshape(_P, _P * _H).astype(jnp.bfloat16)
    pb = proj_b.reshape(1, 1)

    def w_spec(a):
        return pl.BlockSpec(a.shape, lambda i: (0,) * a.ndim)

    operands = [xin, s0, s1] + Ws + [pcol, prow, pb]
    in_specs = [pl.BlockSpec((1, _SEQ, _N, _P * _IN),
                             lambda i: (i, 0, 0, 0))]
    in_specs += [w_spec(a) for a in operands[1:]]

    out = pl.pallas_call(
        _dcrnn_body,
        grid=(_G,),
        in_specs=in_specs,
        out_specs=pl.BlockSpec((1, _HORIZON, _P, _N), lambda i: (i, 0, 0, 0)),
        out_shape=jax.ShapeDtypeStruct((_G, _HORIZON, _P, _N), jnp.float32),
        compiler_params=pltpu.CompilerParams(
            dimension_semantics=("parallel",)),
    )(*operands)
    out = out.transpose(0, 2, 3, 1).reshape(_B, _N, _HORIZON)
    return out[:, None]


# speedup vs baseline: 2.5391x; 1.1819x over previous
import jax
import jax.numpy as jnp
from jax import lax
from jax.experimental import pallas as pl
from jax.experimental.pallas import tpu as pltpu

_B = 64
_N = 325
_IN = 2
_OUT = 1
_H = 64
_ORDER = 2
_HORIZON = 12
_SEQ = 12
_K = 5
_P = 4
_G = _B // _P


def _dcrnn_body(xin_ref, s0_ref, s1_ref, *refs):
    L = [refs[9 * i:9 * (i + 1)] for i in range(4)]
    pcol, prow, pb, out_ref = refs[36:]
    bf = jnp.bfloat16
    s0 = s0_ref[...].astype(bf)
    s1 = s1_ref[...].astype(bf)

    def chain(v):
        v0 = v.astype(bf)
        out = [v0]
        for s in (s0, s1):
            xk = v0
            for _ in range(_ORDER):
                xk = jnp.dot(s, xk,
                             preferred_element_type=jnp.float32).astype(bf)
                out.append(xk)
        return out

    def msum(ks, W_ref, acc):
        for k in range(_K):
            acc = acc + jnp.dot(ks[k], W_ref[k],
                                preferred_element_type=jnp.float32)
        return acc

    def cell(x, h, lw):
        Wrx, Wux, Wcx, Wrh, Wuh, Wch, br, bu, bc = lw
        Xk = chain(x)
        Hk = chain(h)
        r = jax.nn.sigmoid(msum(Hk, Wrh, msum(Xk, Wrx, br[...])))
        u = jax.nn.sigmoid(msum(Hk, Wuh, msum(Xk, Wux, bu[...])))
        Rk = chain(r * h)
        c = jnp.tanh(msum(Rk, Wch, msum(Xk, Wcx, bc[...])))
        return u * h + (1.0 - u) * c

    z = jnp.zeros((_N, _P * _H), dtype=jnp.float32)

    def enc_step(t, hs):
        h0a, h1a, h0b, h1b = hs
        xa = xin_ref[0, t]
        xb = xin_ref[1, t]
        h0a = cell(xa, h0a, L[0])
        h0b = cell(xb, h0b, L[0])
        h1a = cell(h0a, h1a, L[1])
        h1b = cell(h0b, h1b, L[1])
        return (h0a, h1a, h0b, h1b)

    h0a, h1a, h0b, h1b = lax.fori_loop(0, _SEQ, enc_step, (z, z, z, z))

    def proj_store(h1, slot, t):
        hb = h1.astype(bf)
        y_col = jnp.dot(hb, pcol[...],
                        preferred_element_type=jnp.float32) + pb[0, 0]
        y_row = lax.dot_general(prow[...], hb, (((1,), (1,)), ((), ())),
                                preferred_element_type=jnp.float32)
        out_ref[slot, t] = y_row + pb[0, 0]
        return y_col

    def dec_step(t, carry):
        h0a, h1a, xa, h0b, h1b, xb = carry
        h0a = cell(xa, h0a, L[2])
        h0b = cell(xb, h0b, L[2])
        h1a = cell(h0a, h1a, L[3])
        h1b = cell(h0b, h1b, L[3])
        xa = proj_store(h1a, 0, t)
        xb = proj_store(h1b, 1, t)
        return (h0a, h1a, xa, h0b, h1b, xb)

    x0 = jnp.zeros((_N, _P * _OUT), dtype=jnp.float32)
    lax.fori_loop(0, _HORIZON, dec_step, (h0a, h1a, x0, h0b, h1b, x0))


def _bd(M):
    K, a, b = M.shape
    out = jnp.einsum('ij,kab->kiajb', jnp.eye(_P, dtype=M.dtype), M)
    return out.reshape(K, _P * a, _P * b).astype(jnp.bfloat16)


def _layer_weights(Wg, bg, Wc, bc, cin):
    C = cin + _H
    pg = Wg.reshape(C, _K, 2 * _H).transpose(1, 0, 2)
    pc = Wc.reshape(C, _K, _H).transpose(1, 0, 2)
    Wrx = _bd(pg[:, :cin, :_H])
    Wux = _bd(pg[:, :cin, _H:])
    Wcx = _bd(pc[:, :cin, :])
    Wrh = _bd(pg[:, cin:, :_H])
    Wuh = _bd(pg[:, cin:, _H:])
    Wch = _bd(pc[:, cin:, :])
    br = jnp.tile(bg[:_H], _P).reshape(1, -1)
    bu = jnp.tile(bg[_H:], _P).reshape(1, -1)
    bcc = jnp.tile(bc, _P).reshape(1, -1)
    return [Wrx, Wux, Wcx, Wrh, Wuh, Wch, br, bu, bcc]


def kernel(inputs, s0, s1,
           enc0_Wg, enc0_bg, enc0_Wc, enc0_bc,
           enc1_Wg, enc1_bg, enc1_Wc, enc1_bc,
           dec0_Wg, dec0_bg, dec0_Wc, dec0_bc,
           dec1_Wg, dec1_bg, dec1_Wc, dec1_bc,
           proj_W, proj_b):
    xin = (inputs.reshape(_G, _P, _IN, _N, _SEQ)
           .transpose(0, 4, 3, 1, 2).reshape(_G, _SEQ, _N, _P * _IN))
    Ws = []
    for Wg, bg, Wc, bc, cin in ((enc0_Wg, enc0_bg, enc0_Wc, enc0_bc, _IN),
                                (enc1_Wg, enc1_bg, enc1_Wc, enc1_bc, _H),
                                (dec0_Wg, dec0_bg, dec0_Wc, dec0_bc, _OUT),
                                (dec1_Wg, dec1_bg, dec1_Wc, dec1_bc, _H)):
        Ws += _layer_weights(Wg, bg, Wc, bc, cin)
    eyeP = jnp.eye(_P, dtype=jnp.float32)
    pcol = jnp.einsum('ij,ao->iaj', eyeP,
                      proj_W.T).reshape(_P * _H, _P).astype(jnp.bfloat16)
    prow = jnp.einsum('ij,oa->ija', eyeP,
                      proj_W).reshape(_P, _P * _H).astype(jnp.bfloat16)
    pb = proj_b.reshape(1, 1)

    def w_spec(a):
        return pl.BlockSpec(a.shape, lambda i: (0,) * a.ndim)

    operands = [xin, s0, s1] + Ws + [pcol, prow, pb]
    in_specs = [pl.BlockSpec((2, _SEQ, _N, _P * _IN),
                             lambda i: (i, 0, 0, 0))]
    in_specs += [w_spec(a) for a in operands[1:]]

    out = pl.pallas_call(
        _dcrnn_body,
        grid=(_G // 2,),
        in_specs=in_specs,
        out_specs=pl.BlockSpec((2, _HORIZON, _P, _N), lambda i: (i, 0, 0, 0)),
        out_shape=jax.ShapeDtypeStruct((_G, _HORIZON, _P, _N), jnp.float32),
        compiler_params=pltpu.CompilerParams(
            dimension_semantics=("parallel",)),
    )(*operands)
    out = out.transpose(0, 2, 3, 1).reshape(_B, _N, _HORIZON)
    return out[:, None]


# speedup vs baseline: 2.9548x; 1.1637x over previous
import jax
import jax.numpy as jnp
from jax import lax
from jax.experimental import pallas as pl
from jax.experimental.pallas import tpu as pltpu

_B = 64
_N = 325
_IN = 2
_OUT = 1
_H = 64
_ORDER = 2
_HORIZON = 12
_SEQ = 12
_K = 5
_P = 4
_R = 4
_PN = 328
_RN = _R * _PN
_G = _B // (_P * _R)


def _dcrnn_body(xin_ref, s0_ref, s1_ref, *rest):
    Lw = [rest[9 * i:9 * (i + 1)] for i in range(4)]
    pcol, prow, pb, out_ref, xb1, hb, xbe, xbd = rest[36:]
    bf = jnp.bfloat16
    s0 = s0_ref[...]
    s1 = s1_ref[...]

    def chains(buf, w, v):
        vb = v.astype(bf)
        buf[:, 0:w] = vb
        col = w
        for s in (s0, s1):
            cur = [vb[p * _PN:(p + 1) * _PN] for p in range(_R)]
            for _ in range(_ORDER):
                cur = [jnp.dot(s, cp,
                               preferred_element_type=jnp.float32).astype(bf)
                       for cp in cur]
                for p in range(_R):
                    buf[pl.ds(p * _PN, _PN), col:col + w] = cur[p]
                col += w

    def cell(x, h, lw, xb, xw):
        Xr, Xu, Xc, Hr, Hu, Hc, br, bu, bc = lw
        chains(xb, xw, x)
        chains(hb, _P * _H, h)
        xv = xb[...]
        hv = hb[...]
        r = jax.nn.sigmoid(
            jnp.dot(xv, Xr[...], preferred_element_type=jnp.float32)
            + jnp.dot(hv, Hr[...], preferred_element_type=jnp.float32)
            + br[...])
        u = jax.nn.sigmoid(
            jnp.dot(xv, Xu[...], preferred_element_type=jnp.float32)
            + jnp.dot(hv, Hu[...], preferred_element_type=jnp.float32)
            + bu[...])
        chains(hb, _P * _H, r * h)
        rv = hb[...]
        c = jnp.tanh(
            jnp.dot(xv, Xc[...], preferred_element_type=jnp.float32)
            + jnp.dot(rv, Hc[...], preferred_element_type=jnp.float32)
            + bc[...])
        return u * h + (1.0 - u) * c

    z = jnp.zeros((_RN, _P * _H), dtype=jnp.float32)

    def enc_step(t, hs):
        h0, h1 = hs
        x = xin_ref[0, t]
        h0 = cell(x, h0, Lw[0], xbe, _P * _IN)
        h1 = cell(h0, h1, Lw[1], xb1, _P * _H)
        return (h0, h1)

    h0, h1 = lax.fori_loop(0, _SEQ, enc_step, (z, z))

    def dec_step(t, carry):
        h0, h1, x = carry
        h0 = cell(x, h0, Lw[2], xbd, _P * _OUT)
        h1 = cell(h0, h1, Lw[3], xb1, _P * _H)
        h1b = h1.astype(bf)
        y_col = jnp.dot(h1b, pcol[...],
                        preferred_element_type=jnp.float32) + pb[0, 0]
        for p in range(_R):
            y_row = lax.dot_general(
                prow[...], h1b[p * _PN:p * _PN + _N],
                (((1,), (1,)), ((), ())),
                preferred_element_type=jnp.float32)
            out_ref[0, t, p] = y_row + pb[0, 0]
        return (h0, h1, y_col)

    x0 = jnp.zeros((_RN, _P * _OUT), dtype=jnp.float32)
    lax.fori_loop(0, _HORIZON, dec_step, (h0, h1, x0))


def _bd(M):
    K, a, b = M.shape
    out = jnp.einsum('ij,kab->kiajb', jnp.eye(_P, dtype=M.dtype), M)
    return out.reshape(K * _P * a, _P * b).astype(jnp.bfloat16)


def _layer_weights(Wg, bg, Wc, bc, cin):
    C = cin + _H
    pg = Wg.reshape(C, _K, 2 * _H).transpose(1, 0, 2)
    pc = Wc.reshape(C, _K, _H).transpose(1, 0, 2)
    Xr = _bd(pg[:, :cin, :_H])
    Xu = _bd(pg[:, :cin, _H:])
    Xc = _bd(pc[:, :cin, :])
    Hr = _bd(pg[:, cin:, :_H])
    Hu = _bd(pg[:, cin:, _H:])
    Hc = _bd(pc[:, cin:, :])
    br = jnp.tile(bg[:_H], _P).reshape(1, -1)
    bu = jnp.tile(bg[_H:], _P).reshape(1, -1)
    bcc = jnp.tile(bc, _P).reshape(1, -1)
    return [Xr, Xu, Xc, Hr, Hu, Hc, br, bu, bcc]


def kernel(inputs, s0, s1,
           enc0_Wg, enc0_bg, enc0_Wc, enc0_bc,
           enc1_Wg, enc1_bg, enc1_Wc, enc1_bc,
           dec0_Wg, dec0_bg, dec0_Wc, dec0_bc,
           dec1_Wg, dec1_bg, dec1_Wc, dec1_bc,
           proj_W, proj_b):
    xin = (inputs.reshape(_G, _R, _P, _IN, _N, _SEQ)
           .transpose(0, 5, 1, 4, 2, 3)
           .reshape(_G, _SEQ, _R, _N, _P * _IN))
    xin = jnp.pad(xin, ((0, 0), (0, 0), (0, 0), (0, _PN - _N), (0, 0)))
    xin = xin.reshape(_G, _SEQ, _RN, _P * _IN)
    s0p = jnp.pad(s0, ((0, _PN - _N), (0, _PN - _N))).astype(jnp.bfloat16)
    s1p = jnp.pad(s1, ((0, _PN - _N), (0, _PN - _N))).astype(jnp.bfloat16)
    Ws = []
    for Wg, bg, Wc, bc, cin in ((enc0_Wg, enc0_bg, enc0_Wc, enc0_bc, _IN),
                                (enc1_Wg, enc1_bg, enc1_Wc, enc1_bc, _H),
                                (dec0_Wg, dec0_bg, dec0_Wc, dec0_bc, _OUT),
                                (dec1_Wg, dec1_bg, dec1_Wc, dec1_bc, _H)):
        Ws += _layer_weights(Wg, bg, Wc, bc, cin)
    eyeP = jnp.eye(_P, dtype=jnp.float32)
    pcol = jnp.einsum('ij,ao->iaj', eyeP,
                      proj_W.T).reshape(_P * _H, _P).astype(jnp.bfloat16)
    prow = jnp.einsum('ij,oa->ija', eyeP,
                      proj_W).reshape(_P, _P * _H).astype(jnp.bfloat16)
    pb = proj_b.reshape(1, 1)

    def w_spec(a):
        return pl.BlockSpec(a.shape, lambda i: (0,) * a.ndim)

    operands = [xin, s0p, s1p] + Ws + [pcol, prow, pb]
    in_specs = [pl.BlockSpec((1, _SEQ, _RN, _P * _IN),
                             lambda i: (i, 0, 0, 0))]
    in_specs += [w_spec(a) for a in operands[1:]]

    out = pl.pallas_call(
        _dcrnn_body,
        grid=(_G,),
        in_specs=in_specs,
        out_specs=pl.BlockSpec((1, _HORIZON, _R, _P, _N),
                               lambda i: (i, 0, 0, 0, 0)),
        out_shape=jax.ShapeDtypeStruct((_G, _HORIZON, _R, _P, _N),
                                       jnp.float32),
        scratch_shapes=[
            pltpu.VMEM((_RN, _K * _P * _H), jnp.bfloat16),
            pltpu.VMEM((_RN, _K * _P * _H), jnp.bfloat16),
            pltpu.VMEM((_RN, _K * _P * _IN), jnp.bfloat16),
            pltpu.VMEM((_RN, _K * _P * _OUT), jnp.bfloat16),
        ],
        compiler_params=pltpu.CompilerParams(
            dimension_semantics=("parallel",)),
    )(*operands)
    out = out.transpose(0, 2, 3, 4, 1).reshape(_B, _N, _HORIZON)
    return out[:, None]
